# Initial kernel scaffold; baseline (speedup 1.0000x reference)
import jax
import jax.numpy as jnp
from jax.experimental import pallas as pl


def kernel(pairwise_distances, model_prediction_logits, model_true_labels, model_temperature, model_gamma, model_bias):
    raise NotImplementedError("write your pallas kernel here")


# speedup vs baseline: 1.1204x; 1.1204x over previous
import functools

import jax
import jax.numpy as jnp
from jax.experimental import pallas as pl
from jax.experimental.pallas import tpu as pltpu

_K = 32
_ROWS_PER_BLOCK = 8
_LANES = 128


def _select_combine_body(temp_ref, gamma_ref, bias_ref,
                         dist_ref, logits_ref, labels_ref, out_ref, *,
                         n_pad_cols):
    d = dist_ref[...]
    rb = d.shape[0]
    di = jax.lax.bitcast_convert_type(d, jnp.int32)

    def p1_step(i, t):
        cand = t | (1 << (30 - i))
        cnt = jnp.sum((di < cand).astype(jnp.int32), axis=1, keepdims=True)
        return jnp.where(cnt <= _K - 1, cand, t)

    t = jax.lax.fori_loop(0, 31, p1_step, jnp.zeros((rb, 1), jnp.int32))
    c_less = jnp.sum((di < t).astype(jnp.int32), axis=1, keepdims=True)
    m = _K - c_less

    tie = di == t
    idx = jax.lax.broadcasted_iota(jnp.int32, d.shape, 1)

    def p2_step(i, x):
        cand = x | (1 << (17 - i))
        cnt = jnp.sum((tie & (idx < cand)).astype(jnp.int32), axis=1,
                      keepdims=True)
        return jnp.where(cnt <= m - 1, cand, x)

    ix = jax.lax.fori_loop(0, 18, p2_step, jnp.zeros((rb, 1), jnp.int32))

    mask = (di < t) | (tie & (idx <= ix))

    temp = temp_ref[0]
    gamma = gamma_ref[0]
    bias = bias_ref[0]
    v = jnp.tanh(logits_ref[...]) + gamma * labels_ref[...]
    w = jnp.where(mask, jnp.exp(-d / temp), 0.0)
    num = jnp.sum(w * v, axis=1, keepdims=True)
    den = jnp.sum(w, axis=1, keepdims=True)
    res = num / den + bias
    out_ref[...] = jnp.broadcast_to(res, (rb, _LANES))


def _pad_cols(x, n_to, value):
    n = x.shape[-1]
    if n == n_to:
        return x
    widths = [(0, 0)] * (x.ndim - 1) + [(0, n_to - n)]
    return jnp.pad(x, widths, constant_values=value)


@jax.jit
def kernel(pairwise_distances, model_prediction_logits, model_true_labels,
           model_temperature, model_gamma, model_bias):
    batch, sup = pairwise_distances.shape
    n_pad = ((sup + _LANES - 1) // _LANES) * _LANES
    rb = _ROWS_PER_BLOCK
    grid = (batch // rb,)

    d = _pad_cols(pairwise_distances, n_pad, jnp.inf)
    logits = _pad_cols(model_prediction_logits.reshape(1, sup), n_pad, 0.0)
    labels = _pad_cols(model_true_labels.reshape(1, sup), n_pad, 0.0)

    out = pl.pallas_call(
        functools.partial(_select_combine_body, n_pad_cols=n_pad),
        grid=grid,
        in_specs=[
            pl.BlockSpec(memory_space=pltpu.SMEM),
            pl.BlockSpec(memory_space=pltpu.SMEM),
            pl.BlockSpec(memory_space=pltpu.SMEM),
            pl.BlockSpec((rb, n_pad), lambda i: (i, 0)),
            pl.BlockSpec((1, n_pad), lambda i: (0, 0)),
            pl.BlockSpec((1, n_pad), lambda i: (0, 0)),
        ],
        out_specs=pl.BlockSpec((rb, _LANES), lambda i: (i, 0)),
        out_shape=jax.ShapeDtypeStruct((batch, _LANES), jnp.float32),
    )(model_temperature, model_gamma, model_bias, d, logits, labels)
    return out[:, 0]


# speedup vs baseline: 2.9305x; 2.6156x over previous
import functools

import jax
import jax.numpy as jnp
from jax import lax
from jax.experimental import pallas as pl
from jax.experimental.pallas import tpu as pltpu
from jax.experimental.pallas import tpu_sc as plsc

_NC = 2
_NS = 16
_NW = _NC * _NS
_L = 16

_BATCH = 1024
_N = 100000
_K = 32

_ROWS_PER_W = _BATCH // _NW
_CHUNK = 10000
_NCHUNK = _N // _CHUNK
_GROUP = 5
_NGROUP = _CHUNK // (_GROUP * _L)
_CAP = 256
_NBUFV = _CAP // _L


def _scalar_count(mask_i32_vec):
    return plsc.cumsum(mask_i32_vec)[_L - 1]


def _sc_body(dist_hbm, logits_hbm, labels_hbm, scal_hbm, out_hbm,
             dbuf, bufv, bufi, glog, glab, outv, scal_v, sem, semg):
    wid = lax.axis_index("s") * _NC + lax.axis_index("c")
    base_row = wid * _ROWS_PER_W

    pltpu.sync_copy(scal_hbm, scal_v)
    sv = scal_v[pl.ds(0, _L)]
    temp = sv[0]
    gamma = sv[1]
    bias = sv[2]

    iota = lax.iota(jnp.int32, _L)
    inf = jnp.float32(jnp.inf)

    def rebuild(carry):
        off, _ = carry
        vals = []
        bits = []
        valid = []
        for j in range(_NBUFV):
            v = bufv[pl.ds(j * _L, _L)]
            val = (iota + (j * _L)) < off
            v = jnp.where(val, v, inf)
            vals.append(v)
            bits.append(jax.lax.bitcast_convert_type(v, jnp.int32))
            valid.append(val)

        def bit_step(i, t):
            cand = t | (jnp.int32(1) << (30 - i))
            cvec = jnp.full((_L,), cand, jnp.int32)
            cnt = jnp.zeros((_L,), jnp.int32)
            for j in range(_NBUFV):
                cnt = cnt + (bits[j] < cvec).astype(jnp.int32)
            return jnp.where(jnp.sum(cnt) <= _K - 1, cand, t)

        t = lax.fori_loop(0, 31, bit_step, jnp.int32(0))
        tvec = jnp.full((_L,), t, jnp.int32)

        cl = jnp.zeros((_L,), jnp.int32)
        for j in range(_NBUFV):
            cl = cl + (bits[j] < tvec).astype(jnp.int32)
        m = _K - jnp.sum(cl)

        woff = jnp.int32(0)
        ties = jnp.int32(0)
        for j in range(_NBUFV):
            tie = valid[j] & (bits[j] == tvec)
            rank = ties + plsc.cumsum(tie.astype(jnp.int32))
            keep = (bits[j] < tvec) | (tie & (rank <= m))
            iv = bufi[pl.ds(j * _L, _L)]
            plsc.store_compressed(bufv.at[pl.ds(woff, _L)], vals[j],
                                  mask=keep)
            plsc.store_compressed(bufi.at[pl.ds(woff, _L)], iv, mask=keep)
            ties = rank[_L - 1]
            woff = woff + _scalar_count(keep.astype(jnp.int32))

        v0 = bufv[pl.ds(0, _L)]
        v1 = bufv[pl.ds(_L, _L)]
        thr = jnp.maximum(jnp.max(v0), jnp.max(v1))
        return jnp.int32(_K), thr

    def row_body(r, _):
        rowc0 = (base_row + r) * _NCHUNK

        def chunk_body(c, carry):
            pltpu.sync_copy(dist_hbm.at[pl.ds(rowc0 + c, 1)], dbuf)
            cbase = c * _CHUNK

            def group_body(gi, carry):
                off, thr = carry
                ebase = gi * (_GROUP * _L)
                tvec = jnp.full((_L,), thr)
                vs = [dbuf[0, pl.ds(ebase + u * _L, _L)]
                      for u in range(_GROUP)]
                hits = [v < tvec for v in vs]
                anym = hits[0]
                for h in hits[1:]:
                    anym = anym | h
                anyhit = plsc.cumsum(anym.astype(jnp.int32))[_L - 1] > 0

                def slow(carry):
                    off, thr = carry
                    off, thr = lax.cond(off > _CAP - _GROUP * _L,
                                        rebuild, lambda c_: c_, (off, thr))
                    for u in range(_GROUP):
                        idxv = iota + (cbase + ebase + u * _L)
                        plsc.store_compressed(bufv.at[pl.ds(off, _L)],
                                              vs[u], mask=hits[u])
                        plsc.store_compressed(bufi.at[pl.ds(off, _L)],
                                              idxv, mask=hits[u])
                        off = off + _scalar_count(hits[u].astype(jnp.int32))
                    return off, thr

                return lax.cond(anyhit, slow, lambda c_: c_, (off, thr))

            return lax.fori_loop(0, _NGROUP, group_body, carry)

        carry = lax.fori_loop(0, _NCHUNK, chunk_body,
                              (jnp.int32(0), inf))
        off, thr = rebuild(carry)

        idx_ref = bufi.at[pl.ds(0, _K)]
        pltpu.async_copy(logits_hbm.at[idx_ref], glog, semg).wait()
        pltpu.async_copy(labels_hbm.at[idx_ref], glab, semg).wait()

        num = jnp.float32(0.0)
        den = jnp.float32(0.0)
        for j in range(_K // _L):
            d = bufv[pl.ds(j * _L, _L)]
            lg = glog[pl.ds(j * _L, _L)]
            y = glab[pl.ds(j * _L, _L)]
            w = jnp.exp(-d / temp)
            th = 1.0 - 2.0 / (jnp.exp(2.0 * lg) + 1.0)
            num = num + jnp.sum(w * (th + gamma * y))
            den = den + jnp.sum(w)
        resv = jnp.full((_L,), num) / jnp.full((_L,), den) + bias
        plsc.store_scatter(outv, [jnp.full((_L,), r, jnp.int32)],
                           resv, mask=iota == 0)
        return 0

    lax.fori_loop(0, _ROWS_PER_W, row_body, 0)
    pltpu.sync_copy(outv, out_hbm.at[pl.ds(base_row, _ROWS_PER_W)])


@jax.jit
def kernel(pairwise_distances, model_prediction_logits, model_true_labels,
           model_temperature, model_gamma, model_bias):
    dist = pairwise_distances.reshape(_BATCH * _NCHUNK, _CHUNK)
    scal = jnp.concatenate([
        model_temperature.astype(jnp.float32),
        model_gamma.astype(jnp.float32),
        model_bias.astype(jnp.float32),
        jnp.zeros((13,), jnp.float32),
    ])

    mesh = plsc.VectorSubcoreMesh(core_axis_name="c", subcore_axis_name="s",
                                  num_cores=_NC, num_subcores=_NS)
    run = pl.kernel(
        _sc_body,
        out_type=jax.ShapeDtypeStruct((_BATCH,), jnp.float32),
        mesh=mesh,
        compiler_params=pltpu.CompilerParams(needs_layout_passes=False),
        scratch_types=[
            pltpu.VMEM((1, _CHUNK), jnp.float32),
            pltpu.VMEM((_CAP,), jnp.float32),
            pltpu.VMEM((_CAP,), jnp.int32),
            pltpu.VMEM((_K,), jnp.float32),
            pltpu.VMEM((_K,), jnp.float32),
            pltpu.VMEM((_ROWS_PER_W,), jnp.float32),
            pltpu.VMEM((_L,), jnp.float32),
            pltpu.SemaphoreType.DMA,
            pltpu.SemaphoreType.DMA,
        ],
    )
    return run(dist, model_prediction_logits, model_true_labels, scal)


# speedup vs baseline: 3.6498x; 1.2455x over previous
import functools

import jax
import jax.numpy as jnp
from jax import lax
from jax.experimental import pallas as pl
from jax.experimental.pallas import tpu as pltpu
from jax.experimental.pallas import tpu_sc as plsc

_NC = 2
_NS = 16
_NW = _NC * _NS
_L = 16

_BATCH = 1024
_N = 100000
_K = 32

_ROWS_PER_W = _BATCH // _NW
_CHUNK = 10000
_NCHUNK = _N // _CHUNK
_GROUP = 5
_NGROUP = _CHUNK // (_GROUP * _L)
_CAP = 256
_NBUFV = _CAP // _L


def _scalar_count(mask_bool_vec):
    return plsc.all_reduce_population_count(mask_bool_vec)[0]


def _sc_body(dist_hbm, logits_hbm, labels_hbm, scal_hbm, out_hbm,
             dbuf, dbuf1, bufv, bufi, glog, glab, outv, scal_v,
             sem, sem1, semg):
    wid = lax.axis_index("s") * _NC + lax.axis_index("c")
    base_row = wid * _ROWS_PER_W

    pltpu.sync_copy(scal_hbm, scal_v)
    sv = scal_v[pl.ds(0, _L)]
    temp = sv[0]
    gamma = sv[1]
    bias = sv[2]

    iota = lax.iota(jnp.int32, _L)
    inf = jnp.float32(jnp.inf)

    def rebuild(carry):
        off, _ = carry
        vals = []
        bits = []
        valid = []
        for j in range(_NBUFV):
            v = bufv[pl.ds(j * _L, _L)]
            val = (iota + (j * _L)) < off
            v = jnp.where(val, v, inf)
            vals.append(v)
            bits.append(jax.lax.bitcast_convert_type(v, jnp.int32))
            valid.append(val)

        def bit_step(i, t):
            cand = t | (jnp.int32(1) << (30 - i))
            cvec = jnp.full((_L,), cand, jnp.int32)
            cnt = jnp.zeros((_L,), jnp.int32)
            for j in range(_NBUFV):
                cnt = cnt + (bits[j] < cvec).astype(jnp.int32)
            return jnp.where(jnp.sum(cnt) <= _K - 1, cand, t)

        t = lax.fori_loop(0, 31, bit_step, jnp.int32(0))
        tvec = jnp.full((_L,), t, jnp.int32)

        cl = jnp.zeros((_L,), jnp.int32)
        for j in range(_NBUFV):
            cl = cl + (bits[j] < tvec).astype(jnp.int32)
        m = _K - jnp.sum(cl)

        woff = jnp.int32(0)
        ties = jnp.int32(0)
        for j in range(_NBUFV):
            tie = valid[j] & (bits[j] == tvec)
            rank = ties + plsc.cumsum(tie.astype(jnp.int32))
            keep = (bits[j] < tvec) | (tie & (rank <= m))
            iv = bufi[pl.ds(j * _L, _L)]
            plsc.store_compressed(bufv.at[pl.ds(woff, _L)], vals[j],
                                  mask=keep)
            plsc.store_compressed(bufi.at[pl.ds(woff, _L)], iv, mask=keep)
            ties = rank[_L - 1]
            woff = woff + _scalar_count(keep)

        v0 = bufv[pl.ds(0, _L)]
        v1 = bufv[pl.ds(_L, _L)]
        thr = jnp.maximum(jnp.max(v0), jnp.max(v1))
        return jnp.int32(_K), thr

    def scan_chunk(dref, cbase, carry):
        def group_body(gi, carry):
            off, thr = carry
            ebase = gi * (_GROUP * _L)
            tvec = jnp.full((_L,), thr)
            vs = [dref[0, pl.ds(ebase + u * _L, _L)]
                  for u in range(_GROUP)]
            hits = [v < tvec for v in vs]
            anym = hits[0]
            for h in hits[1:]:
                anym = anym | h
            anyhit = _scalar_count(anym) > 0

            def slow(carry):
                off, thr = carry
                off, thr = lax.cond(off > _CAP - _GROUP * _L,
                                    rebuild, lambda c_: c_, (off, thr))
                for u in range(_GROUP):
                    idxv = iota + (cbase + ebase + u * _L)
                    plsc.store_compressed(bufv.at[pl.ds(off, _L)],
                                          vs[u], mask=hits[u])
                    plsc.store_compressed(bufi.at[pl.ds(off, _L)],
                                          idxv, mask=hits[u])
                    off = off + _scalar_count(hits[u])
                return off, thr

            return lax.cond(anyhit, slow, lambda c_: c_, (off, thr))

        return lax.fori_loop(0, _NGROUP, group_body, carry)

    pltpu.async_copy(dist_hbm.at[pl.ds(base_row * _NCHUNK, 1)], dbuf, sem)

    def row_body(r, _):
        rowc0 = (base_row + r) * _NCHUNK

        def pair_body(i, carry):
            c0 = 2 * i
            pltpu.make_async_copy(dist_hbm.at[pl.ds(0, 1)], dbuf, sem).wait()
            pltpu.async_copy(dist_hbm.at[pl.ds(rowc0 + c0 + 1, 1)],
                             dbuf1, sem1)
            carry = scan_chunk(dbuf, c0 * _CHUNK, carry)
            pltpu.make_async_copy(dist_hbm.at[pl.ds(0, 1)],
                                  dbuf1, sem1).wait()
            nxt = jnp.minimum(rowc0 + c0 + 2, _BATCH * _NCHUNK - 1)
            pltpu.async_copy(dist_hbm.at[pl.ds(nxt, 1)], dbuf, sem)
            return scan_chunk(dbuf1, (c0 + 1) * _CHUNK, carry)

        carry = lax.fori_loop(0, _NCHUNK // 2, pair_body,
                              (jnp.int32(0), inf))
        off, thr = rebuild(carry)

        idx_ref = bufi.at[pl.ds(0, _K)]
        pltpu.async_copy(logits_hbm.at[idx_ref], glog, semg).wait()
        pltpu.async_copy(labels_hbm.at[idx_ref], glab, semg).wait()

        num = jnp.float32(0.0)
        den = jnp.float32(0.0)
        for j in range(_K // _L):
            d = bufv[pl.ds(j * _L, _L)]
            lg = glog[pl.ds(j * _L, _L)]
            y = glab[pl.ds(j * _L, _L)]
            w = jnp.exp(-d / temp)
            th = 1.0 - 2.0 / (jnp.exp(2.0 * lg) + 1.0)
            num = num + jnp.sum(w * (th + gamma * y))
            den = den + jnp.sum(w)
        resv = jnp.full((_L,), num) / jnp.full((_L,), den) + bias
        plsc.store_scatter(outv, [jnp.full((_L,), r, jnp.int32)],
                           resv, mask=iota == 0)
        return 0

    lax.fori_loop(0, _ROWS_PER_W, row_body, 0)
    pltpu.make_async_copy(dist_hbm.at[pl.ds(0, 1)], dbuf, sem).wait()
    pltpu.sync_copy(outv, out_hbm.at[pl.ds(base_row, _ROWS_PER_W)])


@jax.jit
def kernel(pairwise_distances, model_prediction_logits, model_true_labels,
           model_temperature, model_gamma, model_bias):
    dist = pairwise_distances.reshape(_BATCH * _NCHUNK, _CHUNK)
    scal = jnp.concatenate([
        model_temperature.astype(jnp.float32),
        model_gamma.astype(jnp.float32),
        model_bias.astype(jnp.float32),
        jnp.zeros((13,), jnp.float32),
    ])

    mesh = plsc.VectorSubcoreMesh(core_axis_name="c", subcore_axis_name="s",
                                  num_cores=_NC, num_subcores=_NS)
    run = pl.kernel(
        _sc_body,
        out_type=jax.ShapeDtypeStruct((_BATCH,), jnp.float32),
        mesh=mesh,
        compiler_params=pltpu.CompilerParams(needs_layout_passes=False),
        scratch_types=[
            pltpu.VMEM((1, _CHUNK), jnp.float32),
            pltpu.VMEM((1, _CHUNK), jnp.float32),
            pltpu.VMEM((_CAP,), jnp.float32),
            pltpu.VMEM((_CAP,), jnp.int32),
            pltpu.VMEM((_K,), jnp.float32),
            pltpu.VMEM((_K,), jnp.float32),
            pltpu.VMEM((_ROWS_PER_W,), jnp.float32),
            pltpu.VMEM((_L,), jnp.float32),
            pltpu.SemaphoreType.DMA,
            pltpu.SemaphoreType.DMA,
            pltpu.SemaphoreType.DMA,
        ],
    )
    return run(dist, model_prediction_logits, model_true_labels, scal)


# speedup vs baseline: 5.0749x; 1.3905x over previous
import functools

import jax
import jax.numpy as jnp
from jax import lax
from jax.experimental import pallas as pl
from jax.experimental.pallas import tpu as pltpu
from jax.experimental.pallas import tpu_sc as plsc

_NC = 2
_NS = 16
_NW = _NC * _NS
_L = 16

_BATCH = 1024
_N = 100000
_K = 32

_ROWS_PER_W = _BATCH // _NW
_CHUNK = 10000
_NCHUNK = _N // _CHUNK
_GROUP = 25
_NGROUP = _CHUNK // (_GROUP * _L)
_CAP = 512
_NBUFV = _CAP // _L


def _scalar_count(mask_bool_vec):
    return plsc.all_reduce_population_count(mask_bool_vec)[0]


def _sc_body(dist_hbm, logits_hbm, labels_hbm, scal_hbm, out_hbm,
             dbuf, dbuf1, bufv, bufi, glog, glab, outv, scal_v,
             sem, sem1, semg):
    wid = lax.axis_index("s") * _NC + lax.axis_index("c")
    base_row = wid * _ROWS_PER_W

    pltpu.sync_copy(scal_hbm, scal_v)
    sv = scal_v[pl.ds(0, _L)]
    temp = sv[0]
    gamma = sv[1]
    bias = sv[2]

    iota = lax.iota(jnp.int32, _L)
    inf = jnp.float32(jnp.inf)

    def rebuild(carry):
        off, _ = carry
        vals = []
        bits = []
        valid = []
        for j in range(_NBUFV):
            v = bufv[pl.ds(j * _L, _L)]
            val = (iota + (j * _L)) < off
            v = jnp.where(val, v, inf)
            vals.append(v)
            bits.append(jax.lax.bitcast_convert_type(v, jnp.int32))
            valid.append(val)

        def bit_step(i, t):
            cand = t | (jnp.int32(1) << (30 - i))
            cvec = jnp.full((_L,), cand, jnp.int32)
            cnt = jnp.zeros((_L,), jnp.int32)
            for j in range(_NBUFV):
                cnt = cnt + (bits[j] < cvec).astype(jnp.int32)
            return jnp.where(jnp.sum(cnt) <= _K - 1, cand, t)

        t = lax.fori_loop(0, 31, bit_step, jnp.int32(0))
        tvec = jnp.full((_L,), t, jnp.int32)

        cl = jnp.zeros((_L,), jnp.int32)
        for j in range(_NBUFV):
            cl = cl + (bits[j] < tvec).astype(jnp.int32)
        m = _K - jnp.sum(cl)

        woff = jnp.int32(0)
        ties = jnp.int32(0)
        for j in range(_NBUFV):
            tie = valid[j] & (bits[j] == tvec)
            rank = ties + plsc.cumsum(tie.astype(jnp.int32))
            keep = (bits[j] < tvec) | (tie & (rank <= m))
            iv = bufi[pl.ds(j * _L, _L)]
            plsc.store_compressed(bufv.at[pl.ds(woff, _L)], vals[j],
                                  mask=keep)
            plsc.store_compressed(bufi.at[pl.ds(woff, _L)], iv, mask=keep)
            ties = rank[_L - 1]
            woff = woff + _scalar_count(keep)

        v0 = bufv[pl.ds(0, _L)]
        v1 = bufv[pl.ds(_L, _L)]
        thr = jnp.maximum(jnp.max(v0), jnp.max(v1))
        return jnp.int32(_K), thr

    def scan_chunk(dref, cbase, carry):
        def group_body(gi, carry):
            off, thr = carry
            ebase = gi * (_GROUP * _L)
            tvec = jnp.full((_L,), thr)
            vs = [dref[0, pl.ds(ebase + u * _L, _L)]
                  for u in range(_GROUP)]
            hits = [v < tvec for v in vs]
            anym = hits[0]
            for h in hits[1:]:
                anym = anym | h
            anyhit = _scalar_count(anym) > 0

            def slow(carry):
                off, thr = carry
                off, thr = lax.cond(off > _CAP - _GROUP * _L,
                                    rebuild, lambda c_: c_, (off, thr))
                for u in range(_GROUP):
                    idxv = iota + (cbase + ebase + u * _L)
                    plsc.store_compressed(bufv.at[pl.ds(off, _L)],
                                          vs[u], mask=hits[u])
                    plsc.store_compressed(bufi.at[pl.ds(off, _L)],
                                          idxv, mask=hits[u])
                    off = off + _scalar_count(hits[u])
                return off, thr

            return lax.cond(anyhit, slow, lambda c_: c_, (off, thr))

        return lax.fori_loop(0, _NGROUP, group_body, carry)

    pltpu.async_copy(dist_hbm.at[pl.ds(base_row * _NCHUNK, 1)], dbuf, sem)

    def row_body(r, _):
        rowc0 = (base_row + r) * _NCHUNK

        def pair_body(i, carry):
            c0 = 2 * i
            pltpu.make_async_copy(dist_hbm.at[pl.ds(0, 1)], dbuf, sem).wait()
            pltpu.async_copy(dist_hbm.at[pl.ds(rowc0 + c0 + 1, 1)],
                             dbuf1, sem1)
            carry = scan_chunk(dbuf, c0 * _CHUNK, carry)
            pltpu.make_async_copy(dist_hbm.at[pl.ds(0, 1)],
                                  dbuf1, sem1).wait()
            nxt = jnp.minimum(rowc0 + c0 + 2, _BATCH * _NCHUNK - 1)
            pltpu.async_copy(dist_hbm.at[pl.ds(nxt, 1)], dbuf, sem)
            return scan_chunk(dbuf1, (c0 + 1) * _CHUNK, carry)

        carry = lax.fori_loop(0, _NCHUNK // 2, pair_body,
                              (jnp.int32(0), inf))
        off, thr = rebuild(carry)

        idx_ref = bufi.at[pl.ds(0, _K)]
        pltpu.async_copy(logits_hbm.at[idx_ref], glog, semg).wait()
        pltpu.async_copy(labels_hbm.at[idx_ref], glab, semg).wait()

        num = jnp.float32(0.0)
        den = jnp.float32(0.0)
        for j in range(_K // _L):
            d = bufv[pl.ds(j * _L, _L)]
            lg = glog[pl.ds(j * _L, _L)]
            y = glab[pl.ds(j * _L, _L)]
            w = jnp.exp(-d / temp)
            th = 1.0 - 2.0 / (jnp.exp(2.0 * lg) + 1.0)
            num = num + jnp.sum(w * (th + gamma * y))
            den = den + jnp.sum(w)
        resv = jnp.full((_L,), num) / jnp.full((_L,), den) + bias
        plsc.store_scatter(outv, [jnp.full((_L,), r, jnp.int32)],
                           resv, mask=iota == 0)
        return 0

    lax.fori_loop(0, _ROWS_PER_W, row_body, 0)
    pltpu.make_async_copy(dist_hbm.at[pl.ds(0, 1)], dbuf, sem).wait()
    pltpu.sync_copy(outv, out_hbm.at[pl.ds(base_row, _ROWS_PER_W)])


@jax.jit
def kernel(pairwise_distances, model_prediction_logits, model_true_labels,
           model_temperature, model_gamma, model_bias):
    dist = pairwise_distances.reshape(_BATCH * _NCHUNK, _CHUNK)
    scal = jnp.concatenate([
        model_temperature.astype(jnp.float32),
        model_gamma.astype(jnp.float32),
        model_bias.astype(jnp.float32),
        jnp.zeros((13,), jnp.float32),
    ])

    mesh = plsc.VectorSubcoreMesh(core_axis_name="c", subcore_axis_name="s",
                                  num_cores=_NC, num_subcores=_NS)
    run = pl.kernel(
        _sc_body,
        out_type=jax.ShapeDtypeStruct((_BATCH,), jnp.float32),
        mesh=mesh,
        compiler_params=pltpu.CompilerParams(needs_layout_passes=False),
        scratch_types=[
            pltpu.VMEM((1, _CHUNK), jnp.float32),
            pltpu.VMEM((1, _CHUNK), jnp.float32),
            pltpu.VMEM((_CAP,), jnp.float32),
            pltpu.VMEM((_CAP,), jnp.int32),
            pltpu.VMEM((_K,), jnp.float32),
            pltpu.VMEM((_K,), jnp.float32),
            pltpu.VMEM((_ROWS_PER_W,), jnp.float32),
            pltpu.VMEM((_L,), jnp.float32),
            pltpu.SemaphoreType.DMA,
            pltpu.SemaphoreType.DMA,
            pltpu.SemaphoreType.DMA,
        ],
    )
    return run(dist, model_prediction_logits, model_true_labels, scal)
